# baseline (device time: 23336 ns/iter reference)
import jax
import jax.numpy as jnp
from jax import lax
from jax.experimental import pallas as pl
from jax.experimental.pallas import tpu as pltpu

N_DEV = 8
B, SQ, DM, HQ_TOT, DH = 2, 256, 512, 32, 64
H_PER = HQ_TOT // N_DEV
BLK = 64
ROWS = B * SQ
CHUNK = ROWS // N_DEV


def _body(x_ref, wq_ref, k_ref, v_ref, wo_ref, out_ref,
          ctx_ref, part_ref, p1_ref, red_ref,
          p1_send, p1_recv, p2_send, p2_recv):
    my = lax.axis_index("i")

    xb = x_ref[...].astype(jnp.bfloat16)
    wq = wq_ref[...].astype(jnp.bfloat16)
    wo = wo_ref[...].astype(jnp.bfloat16)
    q = jnp.dot(xb, wq, preferred_element_type=jnp.float32)
    q = (q * 0.125).astype(jnp.bfloat16)

    r_blk = lax.broadcasted_iota(jnp.int32, (SQ, SQ), 0) // BLK
    c_blk = lax.broadcasted_iota(jnp.int32, (SQ, SQ), 1) // BLK
    mask = r_blk == c_blk

    for b in range(B):
        rows = slice(b * SQ, (b + 1) * SQ)
        for h in range(H_PER):
            cols = slice(h * DH, (h + 1) * DH)
            qh = q[rows, cols]
            scores = lax.dot_general(
                qh, k_ref[h, b], (((1,), (1,)), ((), ())),
                preferred_element_type=jnp.float32)
            scores = jnp.where(mask, scores, -1e9)
            m = jnp.max(scores, axis=1, keepdims=True)
            w = jnp.exp(scores - m)
            w = (w / jnp.sum(w, axis=1, keepdims=True)).astype(jnp.bfloat16)
            ctx = jnp.dot(w, v_ref[h, b],
                          preferred_element_type=jnp.float32)
            ctx_ref[rows, cols] = ctx.astype(jnp.bfloat16)

    barrier = pltpu.get_barrier_semaphore()
    for k in range(1, N_DEV):
        pl.semaphore_signal(barrier, inc=1,
                            device_id=(lax.rem(my + k, N_DEV),),
                            device_id_type=pltpu.DeviceIdType.MESH)
    pl.semaphore_wait(barrier, N_DEV - 1)

    p1 = []
    for k in range(1, N_DEV):
        d = lax.rem(my + k, N_DEV)
        rows = pl.ds(d * CHUNK, CHUNK)
        part_ref[rows, :] = jnp.dot(
            ctx_ref[rows, :], wo,
            preferred_element_type=jnp.float32).astype(jnp.bfloat16)
        rdma = pltpu.make_async_remote_copy(
            src_ref=part_ref.at[rows, :],
            dst_ref=p1_ref.at[k - 1],
            send_sem=p1_send.at[k - 1],
            recv_sem=p1_recv.at[k - 1],
            device_id=(d,),
            device_id_type=pltpu.DeviceIdType.MESH,
        )
        rdma.start()
        p1.append(rdma)

    acc = jnp.dot(ctx_ref[pl.ds(my * CHUNK, CHUNK), :], wo,
                  preferred_element_type=jnp.float32)
    for j in range(N_DEV - 1):
        p1[j].wait_recv()
        acc = acc + p1_ref[j].astype(jnp.float32)
    red_ref[...] = acc.astype(jnp.bfloat16)
    out_ref[pl.ds(my * CHUNK, CHUNK), :] = red_ref[...]

    p2 = []
    for k in range(1, N_DEV):
        d = lax.rem(my + k, N_DEV)
        rdma = pltpu.make_async_remote_copy(
            src_ref=red_ref,
            dst_ref=out_ref.at[pl.ds(my * CHUNK, CHUNK), :],
            send_sem=p2_send.at[k - 1],
            recv_sem=p2_recv.at[k - 1],
            device_id=(d,),
            device_id_type=pltpu.DeviceIdType.MESH,
        )
        rdma.start()
        p2.append(rdma)

    for j in range(N_DEV - 1):
        p1[j].wait_send()

    for j in range(N_DEV - 1):
        sdev = lax.rem(my + N_DEV - (j + 1), N_DEV)
        recv = pltpu.make_async_remote_copy(
            src_ref=red_ref,
            dst_ref=out_ref.at[pl.ds(sdev * CHUNK, CHUNK), :],
            send_sem=p2_send.at[j],
            recv_sem=p2_recv.at[j],
            device_id=(sdev,),
            device_id_type=pltpu.DeviceIdType.MESH,
        )
        recv.wait_recv()

    for j in range(N_DEV - 1):
        p2[j].wait_send()


def kernel(x, Wq, K_ext, V_ext, Wo):
    i = lax.axis_index("i")
    xb = x.reshape(ROWS, DM)
    wq = Wq
    wo = Wo
    zero = jnp.zeros((), jnp.int32)
    k_s = lax.dynamic_slice(K_ext, (zero, zero, i * H_PER, zero),
                            (B, SQ, H_PER, DH)).astype(jnp.bfloat16)
    v_s = lax.dynamic_slice(V_ext, (zero, zero, i * H_PER, zero),
                            (B, SQ, H_PER, DH)).astype(jnp.bfloat16)
    k_my = jnp.transpose(k_s, (2, 0, 1, 3))
    v_my = jnp.transpose(v_s, (2, 0, 1, 3))

    out = pl.pallas_call(
        _body,
        out_shape=jax.ShapeDtypeStruct((ROWS, DM), jnp.bfloat16),
        in_specs=[pl.BlockSpec(memory_space=pltpu.VMEM)] * 5,
        out_specs=pl.BlockSpec(memory_space=pltpu.VMEM),
        scratch_shapes=[
            pltpu.VMEM((ROWS, H_PER * DH), jnp.bfloat16),
            pltpu.VMEM((ROWS, DM), jnp.bfloat16),
            pltpu.VMEM((N_DEV - 1, CHUNK, DM), jnp.bfloat16),
            pltpu.VMEM((CHUNK, DM), jnp.bfloat16),
            pltpu.SemaphoreType.DMA((N_DEV - 1,)),
            pltpu.SemaphoreType.DMA((N_DEV - 1,)),
            pltpu.SemaphoreType.DMA((N_DEV - 1,)),
            pltpu.SemaphoreType.DMA((N_DEV - 1,)),
        ],
        compiler_params=pltpu.CompilerParams(collective_id=0),
    )(xb, wq, k_my, v_my, wo)
    return out.reshape(B, SQ, DM)


# device time: 22884 ns/iter; 1.0198x vs baseline; 1.0198x over previous
import jax
import jax.numpy as jnp
from jax import lax
from jax.experimental import pallas as pl
from jax.experimental.pallas import tpu as pltpu

N_DEV = 8
B, SQ, DM, HQ_TOT, DH = 2, 256, 512, 32, 64
H_PER = HQ_TOT // N_DEV
BLK = 64
ROWS = B * SQ
CHUNK = ROWS // N_DEV
CPB = N_DEV // B


def _attend(q, k_ref, v_ref, ctx_ref, mask, b):
    rows = slice(b * SQ, (b + 1) * SQ)
    for h in range(H_PER):
        cols = slice(h * DH, (h + 1) * DH)
        qh = q[rows, cols]
        scores = lax.dot_general(
            qh, k_ref[h, b], (((1,), (1,)), ((), ())),
            preferred_element_type=jnp.float32)
        scores = jnp.where(mask, scores, -1e9)
        m = jnp.max(scores, axis=1, keepdims=True)
        w = jnp.exp(scores - m)
        w = (w / jnp.sum(w, axis=1, keepdims=True)).astype(jnp.bfloat16)
        ctx = jnp.dot(w, v_ref[h, b], preferred_element_type=jnp.float32)
        ctx_ref[rows, cols] = ctx.astype(jnp.bfloat16)


def _body(x_ref, wq_ref, k_ref, v_ref, wo_ref, out_ref,
          ctx_ref, part_ref, p1_ref, red_ref,
          p1_send, p1_recv, p2_send, p2_recv):
    my = lax.axis_index("i")

    barrier = pltpu.get_barrier_semaphore()
    for k in range(1, N_DEV):
        pl.semaphore_signal(barrier, inc=1,
                            device_id=(lax.rem(my + k, N_DEV),),
                            device_id_type=pltpu.DeviceIdType.MESH)
    pl.semaphore_wait(barrier, N_DEV - 1)

    q = jnp.dot(x_ref[...], wq_ref[...], preferred_element_type=jnp.float32)
    q = (q * 0.125).astype(jnp.bfloat16)

    r_blk = lax.broadcasted_iota(jnp.int32, (SQ, SQ), 0) // BLK
    c_blk = lax.broadcasted_iota(jnp.int32, (SQ, SQ), 1) // BLK
    mask = r_blk == c_blk

    for b in range(B):
        _attend(q, k_ref, v_ref, ctx_ref, mask, b)
        for d in range(b * CPB, (b + 1) * CPB):
            rows = slice(d * CHUNK, (d + 1) * CHUNK)
            part_ref[rows, :] = jnp.dot(
                ctx_ref[rows, :], wo_ref[...],
                preferred_element_type=jnp.float32).astype(jnp.bfloat16)

            @pl.when(my != d)
            def _():
                pltpu.make_async_remote_copy(
                    src_ref=part_ref.at[rows, :],
                    dst_ref=p1_ref.at[my],
                    send_sem=p1_send.at[d],
                    recv_sem=p1_recv.at[my],
                    device_id=(d,),
                    device_id_type=pltpu.DeviceIdType.MESH,
                ).start()

            @pl.when(my == d)
            def _():
                pltpu.make_async_copy(
                    part_ref.at[rows, :], p1_ref.at[my], p1_recv.at[my],
                ).start()

    def _p1_slot(j):
        return pltpu.make_async_remote_copy(
            src_ref=p1_ref.at[j], dst_ref=p1_ref.at[j],
            send_sem=p1_send.at[j], recv_sem=p1_recv.at[j],
            device_id=(my,), device_id_type=pltpu.DeviceIdType.MESH)

    _p1_slot(0).wait_recv()
    acc = p1_ref[0].astype(jnp.float32)
    for j in range(1, N_DEV):
        _p1_slot(j).wait_recv()
        acc = acc + p1_ref[j].astype(jnp.float32)
    red_ref[...] = acc.astype(jnp.bfloat16)

    myrows = pl.ds(my * CHUNK, CHUNK)
    for t in range(N_DEV):
        @pl.when(my != t)
        def _():
            pltpu.make_async_remote_copy(
                src_ref=red_ref,
                dst_ref=out_ref.at[myrows, :],
                send_sem=p2_send.at[t],
                recv_sem=p2_recv.at[my],
                device_id=(t,),
                device_id_type=pltpu.DeviceIdType.MESH,
            ).start()

        @pl.when(my == t)
        def _():
            pltpu.make_async_copy(
                red_ref, out_ref.at[myrows, :], p2_recv.at[my],
            ).start()

    for d in range(N_DEV):
        @pl.when(my != d)
        def _():
            pltpu.make_async_remote_copy(
                src_ref=part_ref.at[pl.ds(d * CHUNK, CHUNK), :],
                dst_ref=p1_ref.at[my],
                send_sem=p1_send.at[d], recv_sem=p1_recv.at[my],
                device_id=(d,), device_id_type=pltpu.DeviceIdType.MESH,
            ).wait_send()

    for j in range(N_DEV):
        pltpu.make_async_remote_copy(
            src_ref=red_ref,
            dst_ref=out_ref.at[pl.ds(j * CHUNK, CHUNK), :],
            send_sem=p2_send.at[j], recv_sem=p2_recv.at[j],
            device_id=(my,), device_id_type=pltpu.DeviceIdType.MESH,
        ).wait_recv()

    for t in range(N_DEV):
        @pl.when(my != t)
        def _():
            pltpu.make_async_remote_copy(
                src_ref=red_ref,
                dst_ref=out_ref.at[myrows, :],
                send_sem=p2_send.at[t], recv_sem=p2_recv.at[my],
                device_id=(t,), device_id_type=pltpu.DeviceIdType.MESH,
            ).wait_send()


def kernel(x, Wq, K_ext, V_ext, Wo):
    i = lax.axis_index("i")
    xb = x.reshape(ROWS, DM).astype(jnp.bfloat16)
    wq = Wq.astype(jnp.bfloat16)
    wo = Wo.astype(jnp.bfloat16)
    zero = jnp.zeros((), jnp.int32)
    k_s = lax.dynamic_slice(K_ext, (zero, zero, i * H_PER, zero),
                            (B, SQ, H_PER, DH)).astype(jnp.bfloat16)
    v_s = lax.dynamic_slice(V_ext, (zero, zero, i * H_PER, zero),
                            (B, SQ, H_PER, DH)).astype(jnp.bfloat16)
    k_my = jnp.transpose(k_s, (2, 0, 1, 3))
    v_my = jnp.transpose(v_s, (2, 0, 1, 3))

    out = pl.pallas_call(
        _body,
        out_shape=jax.ShapeDtypeStruct((ROWS, DM), jnp.bfloat16),
        in_specs=[pl.BlockSpec(memory_space=pltpu.VMEM)] * 5,
        out_specs=pl.BlockSpec(memory_space=pltpu.VMEM),
        scratch_shapes=[
            pltpu.VMEM((ROWS, H_PER * DH), jnp.bfloat16),
            pltpu.VMEM((ROWS, DM), jnp.bfloat16),
            pltpu.VMEM((N_DEV, CHUNK, DM), jnp.bfloat16),
            pltpu.VMEM((CHUNK, DM), jnp.bfloat16),
            pltpu.SemaphoreType.DMA((N_DEV,)),
            pltpu.SemaphoreType.DMA((N_DEV,)),
            pltpu.SemaphoreType.DMA((N_DEV,)),
            pltpu.SemaphoreType.DMA((N_DEV,)),
        ],
        compiler_params=pltpu.CompilerParams(collective_id=0),
    )(xb, wq, k_my, v_my, wo)
    return out.reshape(B, SQ, DM)
